# baseline (speedup 1.0000x reference)
import functools

import jax
import jax.numpy as jnp
from jax import lax
from jax.experimental import pallas as pl
from jax.experimental.pallas import tpu as pltpu
from jax.experimental.pallas import tpu_sc as plsc

NCELLS = 200000
NNODES = 100000
FEAT = 128
NW = 32
CHUNK = 40
NCHUNKS = NCELLS // CHUNK
ITERS = -(-NCHUNKS // NW)
ROW = 393
NGROUP = -(-CHUNK // 16)
GROWS = 128


def _sc_body(u_hbm, t_hbm, ccvp_hbm, tri_hbm, out_hbm,
             idx_v, g_v, ccvp_v, row_v, t_v, isem, gsem, wsem):
    wid = lax.axis_index("c") * 16 + lax.axis_index("s")
    lane = jax.lax.iota(jnp.int32, 16)

    pltpu.sync_copy(t_hbm, t_v)
    tvec = t_v[...]
    for g in range(NGROUP):
        k = lane + 16 * g
        kc = jnp.minimum(k, CHUNK - 1)
        plsc.store_scatter(row_v, [kc, jnp.zeros((16,), jnp.int32)], tvec,
                           mask=k < CHUNK)

    def load_meta(chunk, b):
        pltpu.async_copy(tri_hbm.at[chunk], idx_v.at[b], isem)
        pltpu.async_copy(ccvp_hbm.at[chunk], ccvp_v.at[b], isem)

    def drain_meta(b):
        pltpu.make_async_copy(tri_hbm.at[0], idx_v.at[b], isem).wait()
        pltpu.make_async_copy(ccvp_hbm.at[0], ccvp_v.at[b], isem).wait()

    def start_gather(b):
        pltpu.async_copy(u_hbm.at[idx_v.at[b]], g_v.at[b], gsem)

    def drain_gather(b):
        pltpu.make_async_copy(u_hbm.at[pl.ds(0, GROWS)], g_v.at[b],
                              gsem).wait()

    def drain_write():
        pltpu.make_async_copy(out_hbm.at[pl.ds(0, CHUNK)], row_v,
                              wsem).wait()

    chunk0 = wid
    load_meta(chunk0, 0)
    drain_meta(0)
    start_gather(0)
    load_meta(chunk0 + NW, 1)

    def body(i, carry):
        b = lax.rem(i, 2)
        bn = lax.rem(i + 1, 2)
        chunk = chunk0 + i * NW
        base = chunk * CHUNK

        @pl.when(chunk < NCHUNKS)
        def _():
            drain_gather(b)

            @pl.when(chunk + NW < NCHUNKS)
            def _():
                drain_meta(bn)
                start_gather(bn)

            @pl.when(i > 0)
            def _():
                drain_write()

            bk = jnp.broadcast_to(b, (16,))
            for g in range(NGROUP):
                k = lane + 16 * g
                m = k < CHUNK
                kc = jnp.minimum(k, CHUNK - 1)
                for r in range(2):
                    val = plsc.load_gather(ccvp_v, [bk, 2 * kc + r], mask=m)
                    plsc.store_scatter(row_v, [kc, jnp.full((16,), 1 + r,
                                                            jnp.int32)],
                                       val, mask=m)
                for r in range(6):
                    val = plsc.load_gather(
                        ccvp_v, [bk, 2 * CHUNK + 6 * kc + r], mask=m)
                    plsc.store_scatter(row_v, [kc, jnp.full((16,), 3 + r,
                                                            jnp.int32)],
                                       val, mask=m)

            def cell(c, cc_):
                for v in range(3):
                    for kk in range(FEAT // 16):
                        col = lane + (9 + FEAT * v + 16 * kk)
                        val = g_v[b, 3 * c + v, pl.ds(16 * kk, 16)]
                        plsc.store_scatter(row_v,
                                           [jnp.broadcast_to(c, (16,)), col],
                                           val)
                return cc_

            lax.fori_loop(0, CHUNK, cell, 0)

            pltpu.async_copy(row_v, out_hbm.at[pl.ds(base, CHUNK)], wsem)

            @pl.when(chunk + 2 * NW < NCHUNKS)
            def _():
                load_meta(chunk + 2 * NW, b)

        return carry

    lax.fori_loop(0, ITERS, body, 0)
    drain_write()


@functools.partial(jax.jit, static_argnames=())
def kernel(u, t, cell_centers, cell_local_vertex_pos, triangulation):
    u2 = u.reshape(NNODES, FEAT)
    t16 = jnp.broadcast_to(t.reshape(1), (16,)).astype(jnp.float32)
    tri120 = triangulation.astype(jnp.int32).reshape(NCHUNKS, 3 * CHUNK)
    tri_pad = jnp.concatenate(
        [tri120, jnp.zeros((NCHUNKS, GROWS - 3 * CHUNK), jnp.int32)], axis=1)
    ccvp = jnp.concatenate(
        [cell_centers.reshape(NCHUNKS, 2 * CHUNK),
         cell_local_vertex_pos.reshape(NCHUNKS, 6 * CHUNK)], axis=1)

    mesh = plsc.VectorSubcoreMesh(core_axis_name="c", subcore_axis_name="s")
    out = pl.kernel(
        _sc_body,
        mesh=mesh,
        compiler_params=pltpu.CompilerParams(use_tc_tiling_on_sc=True,
                                             needs_layout_passes=False),
        out_type=jax.ShapeDtypeStruct((NCELLS, ROW), jnp.float32),
        scratch_types=[
            pltpu.VMEM((2, GROWS), jnp.int32),
            pltpu.VMEM((2, GROWS, FEAT), jnp.float32),
            pltpu.VMEM((2, 8 * CHUNK), jnp.float32),
            pltpu.VMEM((CHUNK, ROW), jnp.float32),
            pltpu.VMEM((16,), jnp.float32),
            pltpu.SemaphoreType.DMA,
            pltpu.SemaphoreType.DMA,
            pltpu.SemaphoreType.DMA,
        ],
    )(u2, t16, ccvp, tri_pad)
    return out[None]


# speedup vs baseline: 1.2513x; 1.2513x over previous
import functools

import jax
import jax.numpy as jnp
from jax import lax
from jax.experimental import pallas as pl
from jax.experimental.pallas import tpu as pltpu
from jax.experimental.pallas import tpu_sc as plsc

NCELLS = 200000
NNODES = 100000
FEAT = 128
NW = 32
CHUNK = 40
NCHUNKS = NCELLS // CHUNK
ITERS = -(-NCHUNKS // NW)
ROW = 393
NGROUP = -(-CHUNK // 16)
GROWS = 128


def _sc_body(u_hbm, t_hbm, ccvp_hbm, tri_hbm, out_hbm,
             idx_v, g_v, ccvp_v, row_v, t_v, isem, gsem, wsem):
    wid = lax.axis_index("c") * 16 + lax.axis_index("s")
    lane = jax.lax.iota(jnp.int32, 16)

    pltpu.sync_copy(t_hbm, t_v)
    tvec = t_v[...]
    for g in range(NGROUP):
        k = lane + 16 * g
        kc = jnp.minimum(k, CHUNK - 1)
        plsc.store_scatter(row_v, [kc, jnp.zeros((16,), jnp.int32)], tvec,
                           mask=k < CHUNK)

    def load_meta(chunk, b):
        pltpu.async_copy(tri_hbm.at[chunk], idx_v.at[b], isem)
        pltpu.async_copy(ccvp_hbm.at[chunk], ccvp_v.at[b], isem)

    def drain_meta(b):
        pltpu.make_async_copy(tri_hbm.at[0], idx_v.at[b], isem).wait()
        pltpu.make_async_copy(ccvp_hbm.at[0], ccvp_v.at[b], isem).wait()

    def start_gather(b):
        pltpu.async_copy(u_hbm.at[idx_v.at[b]], g_v.at[b], gsem)

    def drain_gather(b):
        pltpu.make_async_copy(u_hbm.at[pl.ds(0, GROWS)], g_v.at[b],
                              gsem).wait()

    def drain_write():
        pltpu.make_async_copy(out_hbm.at[0, pl.ds(0, CHUNK)], row_v,
                              wsem).wait()

    chunk0 = wid
    load_meta(chunk0, 0)
    drain_meta(0)
    start_gather(0)
    load_meta(chunk0 + NW, 1)

    def body(i, carry):
        b = lax.rem(i, 2)
        bn = lax.rem(i + 1, 2)
        chunk = chunk0 + i * NW
        base = chunk * CHUNK

        @pl.when(chunk < NCHUNKS)
        def _():
            drain_gather(b)

            @pl.when(chunk + NW < NCHUNKS)
            def _():
                drain_meta(bn)
                start_gather(bn)

            @pl.when(i > 0)
            def _():
                drain_write()

            bk = jnp.broadcast_to(b, (16,))
            for g in range(NGROUP):
                k = lane + 16 * g
                m = k < CHUNK
                kc = jnp.minimum(k, CHUNK - 1)
                for r in range(2):
                    val = plsc.load_gather(ccvp_v, [bk, 2 * kc + r], mask=m)
                    plsc.store_scatter(row_v, [kc, jnp.full((16,), 1 + r,
                                                            jnp.int32)],
                                       val, mask=m)
                for r in range(6):
                    val = plsc.load_gather(
                        ccvp_v, [bk, 2 * CHUNK + 6 * kc + r], mask=m)
                    plsc.store_scatter(row_v, [kc, jnp.full((16,), 3 + r,
                                                            jnp.int32)],
                                       val, mask=m)

            def cell(c, cc_):
                for v in range(3):
                    for kk in range(FEAT // 16):
                        col = lane + (9 + FEAT * v + 16 * kk)
                        val = g_v[b, 3 * c + v, pl.ds(16 * kk, 16)]
                        plsc.store_scatter(row_v,
                                           [jnp.broadcast_to(c, (16,)), col],
                                           val)
                return cc_

            lax.fori_loop(0, CHUNK, cell, 0)

            pltpu.async_copy(row_v, out_hbm.at[0, pl.ds(base, CHUNK)], wsem)

            @pl.when(chunk + 2 * NW < NCHUNKS)
            def _():
                load_meta(chunk + 2 * NW, b)

        return carry

    lax.fori_loop(0, ITERS, body, 0)
    drain_write()


@functools.partial(jax.jit, static_argnames=())
def kernel(u, t, cell_centers, cell_local_vertex_pos, triangulation):
    u2 = u.reshape(NNODES, FEAT)
    t16 = jnp.broadcast_to(t.reshape(1), (16,)).astype(jnp.float32)
    tri120 = triangulation.astype(jnp.int32).reshape(NCHUNKS, 3 * CHUNK)
    tri_pad = jnp.concatenate(
        [tri120, jnp.zeros((NCHUNKS, GROWS - 3 * CHUNK), jnp.int32)], axis=1)
    ccvp = jnp.concatenate(
        [cell_centers.reshape(NCHUNKS, 2 * CHUNK),
         cell_local_vertex_pos.reshape(NCHUNKS, 6 * CHUNK)], axis=1)

    mesh = plsc.VectorSubcoreMesh(core_axis_name="c", subcore_axis_name="s")
    out = pl.kernel(
        _sc_body,
        mesh=mesh,
        compiler_params=pltpu.CompilerParams(use_tc_tiling_on_sc=True,
                                             needs_layout_passes=False),
        out_type=jax.ShapeDtypeStruct((1, NCELLS, ROW), jnp.float32),
        scratch_types=[
            pltpu.VMEM((2, GROWS), jnp.int32),
            pltpu.VMEM((2, GROWS, FEAT), jnp.float32),
            pltpu.VMEM((2, 8 * CHUNK), jnp.float32),
            pltpu.VMEM((CHUNK, ROW), jnp.float32),
            pltpu.VMEM((16,), jnp.float32),
            pltpu.SemaphoreType.DMA,
            pltpu.SemaphoreType.DMA,
            pltpu.SemaphoreType.DMA,
        ],
    )(u2, t16, ccvp, tri_pad)
    return out


# speedup vs baseline: 1.2588x; 1.0060x over previous
import functools

import jax
import jax.numpy as jnp
from jax import lax
from jax.experimental import pallas as pl
from jax.experimental.pallas import tpu as pltpu
from jax.experimental.pallas import tpu_sc as plsc

NCELLS = 200000
NNODES = 100000
FEAT = 128
NW = 32
CHUNK = 40
NCHUNKS = NCELLS // CHUNK
ITERS = -(-NCHUNKS // NW)
ROW = 393
NGROUP = -(-CHUNK // 16)
GROWS = 128
TAIL = ROW - 3 * FEAT


def _sc_body(u_hbm, t_hbm, ccvp_hbm, tri_hbm, out_hbm,
             idx_v, g_v, ccvp_v, rt0_v, rt1_v, rt2_v, rt3_v, t_v,
             isem, gsem, wsem):
    wid = lax.axis_index("c") * 16 + lax.axis_index("s")
    lane = jax.lax.iota(jnp.int32, 16)

    pltpu.sync_copy(t_hbm, t_v)
    tvec = t_v[...]
    for g in range(NGROUP):
        k = lane + 16 * g
        kc = jnp.minimum(k, CHUNK - 1)
        plsc.store_scatter(rt0_v, [kc, jnp.zeros((16,), jnp.int32)], tvec,
                           mask=k < CHUNK)

    def load_meta(chunk, b):
        pltpu.async_copy(tri_hbm.at[chunk], idx_v.at[b], isem)
        pltpu.async_copy(ccvp_hbm.at[chunk], ccvp_v.at[b], isem)

    def drain_meta(b):
        pltpu.make_async_copy(tri_hbm.at[0], idx_v.at[b], isem).wait()
        pltpu.make_async_copy(ccvp_hbm.at[0], ccvp_v.at[b], isem).wait()

    def start_gather(b):
        pltpu.async_copy(u_hbm.at[idx_v.at[b]], g_v.at[b], gsem)

    def drain_gather(b):
        pltpu.make_async_copy(u_hbm.at[pl.ds(0, GROWS)], g_v.at[b],
                              gsem).wait()

    def out_slices(base):
        yield rt0_v, out_hbm.at[0, pl.ds(base, CHUNK), pl.ds(0, FEAT)]
        yield rt1_v, out_hbm.at[0, pl.ds(base, CHUNK), pl.ds(FEAT, FEAT)]
        yield rt2_v, out_hbm.at[0, pl.ds(base, CHUNK), pl.ds(2 * FEAT, FEAT)]
        yield rt3_v, out_hbm.at[0, pl.ds(base, CHUNK), pl.ds(3 * FEAT, TAIL)]

    def start_write(base):
        for src, dst in out_slices(base):
            pltpu.async_copy(src, dst, wsem)

    def drain_write():
        for src, dst in out_slices(0):
            pltpu.make_async_copy(dst, src, wsem).wait()

    chunk0 = wid
    load_meta(chunk0, 0)
    drain_meta(0)
    start_gather(0)
    load_meta(chunk0 + NW, 1)

    def body(i, carry):
        b = lax.rem(i, 2)
        bn = lax.rem(i + 1, 2)
        chunk = chunk0 + i * NW
        base = chunk * CHUNK

        @pl.when(chunk < NCHUNKS)
        def _():
            drain_gather(b)

            @pl.when(chunk + NW < NCHUNKS)
            def _():
                drain_meta(bn)
                start_gather(bn)

            @pl.when(i > 0)
            def _():
                drain_write()

            bk = jnp.broadcast_to(b, (16,))
            for g in range(NGROUP):
                k = lane + 16 * g
                m = k < CHUNK
                kc = jnp.minimum(k, CHUNK - 1)
                for r in range(2):
                    val = plsc.load_gather(ccvp_v, [bk, 2 * kc + r], mask=m)
                    plsc.store_scatter(rt0_v, [kc, jnp.full((16,), 1 + r,
                                                            jnp.int32)],
                                       val, mask=m)
                for r in range(6):
                    val = plsc.load_gather(
                        ccvp_v, [bk, 2 * CHUNK + 6 * kc + r], mask=m)
                    plsc.store_scatter(rt0_v, [kc, jnp.full((16,), 3 + r,
                                                            jnp.int32)],
                                       val, mask=m)

            lo_mask = lane < (FEAT - TAIL - 7 * 16)
            hi_mask = jnp.logical_not(lo_mask)
            tail_col = lane + (TAIL + 7 * 16)
            head_col = lane - 7

            def cell(c, cc_):
                ck = jnp.broadcast_to(c, (16,))
                for v, rt in ((0, rt0_v), (1, rt1_v), (2, rt2_v)):
                    for j in range(7):
                        rt[c, pl.ds(TAIL + 16 * j, 16)] = (
                            g_v[b, 3 * c + v, pl.ds(16 * j, 16)])
                    bv = g_v[b, 3 * c + v, pl.ds(112, 16)]
                    nxt = (rt1_v, rt2_v, rt3_v)[v]
                    plsc.store_scatter(rt, [ck, tail_col], bv, mask=lo_mask)
                    plsc.store_scatter(nxt, [ck, head_col], bv, mask=hi_mask)
                return cc_

            lax.fori_loop(0, CHUNK, cell, 0)

            start_write(base)

            @pl.when(chunk + 2 * NW < NCHUNKS)
            def _():
                load_meta(chunk + 2 * NW, b)

        return carry

    lax.fori_loop(0, ITERS, body, 0)
    drain_write()


@functools.partial(jax.jit, static_argnames=())
def kernel(u, t, cell_centers, cell_local_vertex_pos, triangulation):
    u2 = u.reshape(NNODES, FEAT)
    t16 = jnp.broadcast_to(t.reshape(1), (16,)).astype(jnp.float32)
    tri120 = triangulation.astype(jnp.int32).reshape(NCHUNKS, 3 * CHUNK)
    tri_pad = jnp.concatenate(
        [tri120, jnp.zeros((NCHUNKS, GROWS - 3 * CHUNK), jnp.int32)], axis=1)
    ccvp = jnp.concatenate(
        [cell_centers.reshape(NCHUNKS, 2 * CHUNK),
         cell_local_vertex_pos.reshape(NCHUNKS, 6 * CHUNK)], axis=1)

    mesh = plsc.VectorSubcoreMesh(core_axis_name="c", subcore_axis_name="s")
    out = pl.kernel(
        _sc_body,
        mesh=mesh,
        compiler_params=pltpu.CompilerParams(use_tc_tiling_on_sc=True,
                                             needs_layout_passes=False),
        out_type=jax.ShapeDtypeStruct((1, NCELLS, ROW), jnp.float32),
        scratch_types=[
            pltpu.VMEM((2, GROWS), jnp.int32),
            pltpu.VMEM((2, GROWS, FEAT), jnp.float32),
            pltpu.VMEM((2, 8 * CHUNK), jnp.float32),
            pltpu.VMEM((CHUNK, FEAT), jnp.float32),
            pltpu.VMEM((CHUNK, FEAT), jnp.float32),
            pltpu.VMEM((CHUNK, FEAT), jnp.float32),
            pltpu.VMEM((CHUNK, TAIL), jnp.float32),
            pltpu.VMEM((16,), jnp.float32),
            pltpu.SemaphoreType.DMA,
            pltpu.SemaphoreType.DMA,
            pltpu.SemaphoreType.DMA,
        ],
    )(u2, t16, ccvp, tri_pad)
    return out


# speedup vs baseline: 1.3453x; 1.0687x over previous
import functools

import jax
import jax.numpy as jnp
from jax import lax
from jax.experimental import pallas as pl
from jax.experimental.pallas import tpu as pltpu
from jax.experimental.pallas import tpu_sc as plsc

NCELLS = 200000
NNODES = 100000
FEAT = 128
NW = 32
CHUNK = 40
NCHUNKS = NCELLS // CHUNK
ITERS = -(-NCHUNKS // NW)
ROW = 393
NGROUP = -(-CHUNK // 16)
GROWS = 128
TAIL = ROW - 3 * FEAT


def _sc_body(u_hbm, t_hbm, ccvp_hbm, tri_hbm, out_hbm,
             idx_v, g_v, ccvp_v, rt0_v, rt1_v, rt2_v, rt3_v,
             t_v, isem, gsem, wsem):
    wid = lax.axis_index("c") * 16 + lax.axis_index("s")
    lane = jax.lax.iota(jnp.int32, 16)

    pltpu.sync_copy(t_hbm, t_v)
    tvec = t_v[...]
    for g in range(NGROUP):
        k = lane + 16 * g
        kc = jnp.minimum(k, CHUNK - 1)
        plsc.store_scatter(rt0_v, [kc, jnp.zeros((16,), jnp.int32)], tvec,
                           mask=k < CHUNK)

    def load_meta(chunk, b):
        pltpu.async_copy(tri_hbm.at[chunk], idx_v.at[b], isem)
        pltpu.async_copy(ccvp_hbm.at[chunk], ccvp_v.at[b], isem)

    def drain_meta(b):
        pltpu.make_async_copy(tri_hbm.at[0], idx_v.at[b], isem).wait()
        pltpu.make_async_copy(ccvp_hbm.at[0], ccvp_v.at[b], isem).wait()

    def start_gather(b):
        pltpu.async_copy(u_hbm.at[idx_v.at[b]], g_v.at[b], gsem)

    def drain_gather(b):
        pltpu.make_async_copy(u_hbm.at[pl.ds(0, GROWS)], g_v.at[b],
                              gsem).wait()

    def out_slices(base):
        yield rt0_v, out_hbm.at[0, pl.ds(base, CHUNK), pl.ds(0, FEAT)]
        yield rt1_v, out_hbm.at[0, pl.ds(base, CHUNK), pl.ds(FEAT, FEAT)]
        yield rt2_v, out_hbm.at[0, pl.ds(base, CHUNK), pl.ds(2 * FEAT, FEAT)]
        yield rt3_v, out_hbm.at[0, pl.ds(base, CHUNK), pl.ds(3 * FEAT, TAIL)]

    def start_write(base):
        for src, dst in out_slices(base):
            pltpu.async_copy(src, dst, wsem)

    def drain_write():
        for src, dst in out_slices(0):
            pltpu.make_async_copy(dst, src, wsem).wait()

    chunk0 = wid
    load_meta(chunk0, 0)
    drain_meta(0)
    start_gather(0)
    load_meta(chunk0 + NW, 1)

    def body(i, carry):
        b = lax.rem(i, 2)
        bn = lax.rem(i + 1, 2)
        chunk = chunk0 + i * NW
        base = chunk * CHUNK

        @pl.when(chunk < NCHUNKS)
        def _():
            drain_gather(b)

            @pl.when(chunk + NW < NCHUNKS)
            def _():
                drain_meta(bn)
                start_gather(bn)

            @pl.when(i > 0)
            def _():
                drain_write()

            bk = jnp.broadcast_to(b, (16,))
            for g in range(NGROUP):
                k = lane + 16 * g
                m = k < CHUNK
                kc = jnp.minimum(k, CHUNK - 1)
                for r in range(2):
                    val = plsc.load_gather(ccvp_v, [bk, 2 * kc + r], mask=m)
                    plsc.store_scatter(rt0_v, [kc, jnp.full((16,), 1 + r,
                                                            jnp.int32)],
                                       val, mask=m)
                for r in range(6):
                    col = 2 * CHUNK * (1 + r // 2) + 2 * kc + (r % 2)
                    val = plsc.load_gather(ccvp_v, [bk, col], mask=m)
                    plsc.store_scatter(rt0_v, [kc, jnp.full((16,), 3 + r,
                                                            jnp.int32)],
                                       val, mask=m)

            lo_mask = lane < (FEAT - TAIL - 7 * 16)
            hi_mask = jnp.logical_not(lo_mask)
            tail_col = lane + (TAIL + 7 * 16)
            head_col = lane - 7

            def cell(c, cc_):
                ck = jnp.broadcast_to(c, (16,))
                for v, rt in ((0, rt0_v), (1, rt1_v), (2, rt2_v)):
                    for j in range(7):
                        rt[c, pl.ds(TAIL + 16 * j, 16)] = (
                            g_v[b, 3 * c + v, pl.ds(16 * j, 16)])
                    bv = g_v[b, 3 * c + v, pl.ds(112, 16)]
                    nxt = (rt1_v, rt2_v, rt3_v)[v]
                    plsc.store_scatter(rt, [ck, tail_col], bv, mask=lo_mask)
                    plsc.store_scatter(nxt, [ck, head_col], bv, mask=hi_mask)
                return cc_

            lax.fori_loop(0, CHUNK, cell, 0)

            start_write(base)

            @pl.when(chunk + 2 * NW < NCHUNKS)
            def _():
                load_meta(chunk + 2 * NW, b)

        return carry

    lax.fori_loop(0, ITERS, body, 0)
    drain_write()


@functools.partial(jax.jit, static_argnames=())
def kernel(u, t, cell_centers, cell_local_vertex_pos, triangulation):
    u2 = u.reshape(NNODES, FEAT)
    t16 = jnp.broadcast_to(t.reshape(1), (16,)).astype(jnp.float32)
    tri120 = triangulation.astype(jnp.int32).reshape(NCHUNKS, 3 * CHUNK)
    tri_pad = jnp.concatenate(
        [tri120, jnp.zeros((NCHUNKS, GROWS - 3 * CHUNK), jnp.int32)], axis=1)
    ccvp = jnp.concatenate(
        [cell_centers.reshape(NCHUNKS, 2 * CHUNK)]
        + [cell_local_vertex_pos[:, v, :].reshape(NCHUNKS, 2 * CHUNK)
           for v in range(3)], axis=1)

    mesh = plsc.VectorSubcoreMesh(core_axis_name="c", subcore_axis_name="s")
    out = pl.kernel(
        _sc_body,
        mesh=mesh,
        compiler_params=pltpu.CompilerParams(use_tc_tiling_on_sc=True,
                                             needs_layout_passes=False),
        out_type=jax.ShapeDtypeStruct((1, NCELLS, ROW), jnp.float32),
        scratch_types=[
            pltpu.VMEM((2, GROWS), jnp.int32),
            pltpu.VMEM((2, GROWS, FEAT), jnp.float32),
            pltpu.VMEM((2, 8 * CHUNK), jnp.float32),
            pltpu.VMEM((CHUNK, FEAT), jnp.float32),
            pltpu.VMEM((CHUNK, FEAT), jnp.float32),
            pltpu.VMEM((CHUNK, FEAT), jnp.float32),
            pltpu.VMEM((CHUNK, TAIL), jnp.float32),
            pltpu.VMEM((16,), jnp.float32),
            pltpu.SemaphoreType.DMA,
            pltpu.SemaphoreType.DMA,
            pltpu.SemaphoreType.DMA,
        ],
    )(u2, t16, ccvp, tri_pad)
    return out


# speedup vs baseline: 1.3463x; 1.0007x over previous
import functools

import jax
import jax.numpy as jnp
from jax import lax
from jax.experimental import pallas as pl
from jax.experimental.pallas import tpu as pltpu
from jax.experimental.pallas import tpu_sc as plsc

NCELLS = 200000
NNODES = 100000
FEAT = 128
NW = 32
CHUNK = 40
NCHUNKS = NCELLS // CHUNK
ITERS = -(-NCHUNKS // NW)
ROW = 393
NGROUP = -(-CHUNK // 16)
GROWS = 128
TAIL = ROW - 3 * FEAT


def _sc_body(u_hbm, t_hbm, ccvp_hbm, tri_hbm, out_hbm,
             idx_v, g_v, ccvp_v, rt_v, rt3_v,
             t_v, isem, gsem, wsem):
    wid = lax.axis_index("c") * 16 + lax.axis_index("s")
    lane = jax.lax.iota(jnp.int32, 16)

    pltpu.sync_copy(t_hbm, t_v)
    tvec = t_v[...]
    for g in range(NGROUP):
        k = lane + 16 * g
        kc = jnp.minimum(k, CHUNK - 1)
        plsc.store_scatter(rt_v, [jnp.zeros((16,), jnp.int32), kc,
                                  jnp.zeros((16,), jnp.int32)], tvec,
                           mask=k < CHUNK)

    def load_meta(chunk, b):
        pltpu.async_copy(tri_hbm.at[chunk], idx_v.at[b], isem)
        pltpu.async_copy(ccvp_hbm.at[chunk], ccvp_v.at[b], isem)

    def drain_meta(b):
        pltpu.make_async_copy(tri_hbm.at[0], idx_v.at[b], isem).wait()
        pltpu.make_async_copy(ccvp_hbm.at[0], ccvp_v.at[b], isem).wait()

    def start_gather(b):
        pltpu.async_copy(u_hbm.at[idx_v.at[b]], g_v.at[b], gsem)

    def drain_gather(b):
        pltpu.make_async_copy(u_hbm.at[pl.ds(0, GROWS)], g_v.at[b],
                              gsem).wait()

    def out_slices(base):
        for tile in range(3):
            yield (rt_v.at[tile],
                   out_hbm.at[0, pl.ds(base, CHUNK),
                              pl.ds(tile * FEAT, FEAT)])
        yield rt3_v, out_hbm.at[0, pl.ds(base, CHUNK), pl.ds(3 * FEAT, TAIL)]

    def start_write(base):
        for src, dst in out_slices(base):
            pltpu.async_copy(src, dst, wsem)

    def drain_write():
        for src, dst in out_slices(0):
            pltpu.make_async_copy(dst, src, wsem).wait()

    chunk0 = wid
    load_meta(chunk0, 0)
    drain_meta(0)
    start_gather(0)
    load_meta(chunk0 + NW, 1)

    def body(i, carry):
        b = lax.rem(i, 2)
        bn = lax.rem(i + 1, 2)
        chunk = chunk0 + i * NW
        base = chunk * CHUNK

        @pl.when(chunk < NCHUNKS)
        def _():
            drain_gather(b)

            @pl.when(chunk + NW < NCHUNKS)
            def _():
                drain_meta(bn)
                start_gather(bn)

            @pl.when(i > 0)
            def _():
                drain_write()

            bk = jnp.broadcast_to(b, (16,))
            for g in range(NGROUP):
                k = lane + 16 * g
                m = k < CHUNK
                kc = jnp.minimum(k, CHUNK - 1)
                zv = jnp.zeros((16,), jnp.int32)
                for r in range(2):
                    val = plsc.load_gather(ccvp_v, [bk, 2 * kc + r], mask=m)
                    plsc.store_scatter(rt_v, [zv, kc,
                                              jnp.full((16,), 1 + r,
                                                       jnp.int32)],
                                       val, mask=m)
                for r in range(6):
                    col = 2 * CHUNK * (1 + r // 2) + 2 * kc + (r % 2)
                    val = plsc.load_gather(ccvp_v, [bk, col], mask=m)
                    plsc.store_scatter(rt_v, [zv, kc,
                                              jnp.full((16,), 3 + r,
                                                       jnp.int32)],
                                       val, mask=m)

            lo_mask = lane < (FEAT - TAIL - 7 * 16)
            hi_mask = jnp.logical_not(lo_mask)
            tail_col = lane + (TAIL + 7 * 16)
            head_col = lane - 7
            bnd_col = jnp.where(lo_mask, tail_col, head_col)

            def cell(c, cc_):
                ck = jnp.broadcast_to(c, (16,))
                for v in range(3):
                    for j in range(7):
                        rt_v[v, c, pl.ds(TAIL + 16 * j, 16)] = (
                            g_v[b, 3 * c + v, pl.ds(16 * j, 16)])
                    bv = g_v[b, 3 * c + v, pl.ds(112, 16)]
                    if v < 2:
                        tv = jnp.where(lo_mask, v, v + 1)
                        plsc.store_scatter(rt_v, [tv, ck, bnd_col], bv)
                    else:
                        plsc.store_scatter(rt_v, [jnp.full((16,), 2,
                                                           jnp.int32),
                                                  ck, tail_col],
                                           bv, mask=lo_mask)
                        plsc.store_scatter(rt3_v, [ck, head_col], bv,
                                           mask=hi_mask)
                return cc_

            lax.fori_loop(0, CHUNK, cell, 0)

            start_write(base)

            @pl.when(chunk + 2 * NW < NCHUNKS)
            def _():
                load_meta(chunk + 2 * NW, b)

        return carry

    lax.fori_loop(0, ITERS, body, 0)
    drain_write()


@functools.partial(jax.jit, static_argnames=())
def kernel(u, t, cell_centers, cell_local_vertex_pos, triangulation):
    u2 = u.reshape(NNODES, FEAT)
    t16 = jnp.broadcast_to(t.reshape(1), (16,)).astype(jnp.float32)
    tri120 = triangulation.astype(jnp.int32).reshape(NCHUNKS, 3 * CHUNK)
    tri_pad = jnp.concatenate(
        [tri120, jnp.zeros((NCHUNKS, GROWS - 3 * CHUNK), jnp.int32)], axis=1)
    ccvp = jnp.concatenate(
        [cell_centers.reshape(NCHUNKS, 2 * CHUNK)]
        + [cell_local_vertex_pos[:, v, :].reshape(NCHUNKS, 2 * CHUNK)
           for v in range(3)], axis=1)

    mesh = plsc.VectorSubcoreMesh(core_axis_name="c", subcore_axis_name="s")
    out = pl.kernel(
        _sc_body,
        mesh=mesh,
        compiler_params=pltpu.CompilerParams(use_tc_tiling_on_sc=True,
                                             needs_layout_passes=False),
        out_type=jax.ShapeDtypeStruct((1, NCELLS, ROW), jnp.float32),
        scratch_types=[
            pltpu.VMEM((2, GROWS), jnp.int32),
            pltpu.VMEM((2, GROWS, FEAT), jnp.float32),
            pltpu.VMEM((2, 8 * CHUNK), jnp.float32),
            pltpu.VMEM((3, CHUNK, FEAT), jnp.float32),
            pltpu.VMEM((CHUNK, TAIL), jnp.float32),
            pltpu.VMEM((16,), jnp.float32),
            pltpu.SemaphoreType.DMA,
            pltpu.SemaphoreType.DMA,
            pltpu.SemaphoreType.DMA,
        ],
    )(u2, t16, ccvp, tri_pad)
    return out
